# speedup vs baseline: 5.5237x; 5.5237x over previous
import functools

import jax
import jax.numpy as jnp
from jax import lax
from jax.experimental import pallas as pl
from jax.experimental.pallas import tpu as pltpu
from jax.experimental.pallas import tpu_sc as plsc

CODEBOOK_SIZE = 8192
CODEBOOK_DIM = 256
N_TOKENS = 262144

NUM_CORES = 2
NUM_SUBCORES = 16
NUM_WORKERS = NUM_CORES * NUM_SUBCORES
B_PER_W = N_TOKENS // NUM_WORKERS
CHUNK = 128
NCHUNK = B_PER_W // CHUNK

_MESH = plsc.VectorSubcoreMesh(core_axis_name="c", subcore_axis_name="s")


@functools.partial(
    pl.kernel,
    mesh=_MESH,
    out_type=jax.ShapeDtypeStruct((N_TOKENS, CODEBOOK_DIM), jnp.float32),
    scratch_types=[
        pltpu.VMEM((NCHUNK, CHUNK), jnp.int32),
        pltpu.VMEM((2, CHUNK, CODEBOOK_DIM), jnp.float32),
        pltpu.SemaphoreType.DMA,
        pltpu.SemaphoreType.DMA,
        pltpu.SemaphoreType.DMA,
        pltpu.SemaphoreType.DMA,
    ],
)
def _codebook_gather(weight_hbm, idx_hbm, out_hbm, idx_v, rows_v,
                     gsem0, gsem1, ssem0, ssem1):
    wid = lax.axis_index("s") * NUM_CORES + lax.axis_index("c")
    base = wid * B_PER_W
    gsems = [gsem0, gsem1]
    ssems = [ssem0, ssem1]

    pltpu.sync_copy(idx_hbm.at[wid], idx_v)

    def start_gather(g, buf):
        pltpu.make_async_copy(
            weight_hbm.at[idx_v.at[g]], rows_v.at[buf], gsems[buf]).start()

    def wait_gather(buf):
        pltpu.make_async_copy(
            weight_hbm.at[idx_v.at[0]], rows_v.at[buf], gsems[buf]).wait()

    def start_store(g, buf):
        pltpu.make_async_copy(
            rows_v.at[buf], out_hbm.at[pl.ds(base + g * CHUNK, CHUNK)],
            ssems[buf]).start()

    def wait_store(buf):
        pltpu.make_async_copy(
            rows_v.at[buf], out_hbm.at[pl.ds(base, CHUNK)], ssems[buf]).wait()

    start_gather(0, 0)
    start_gather(1, 1)
    wait_gather(0)
    start_store(0, 0)

    def steady(i, carry):
        for b in (1, 0):
            g = 1 + 2 * i + (1 - b)
            nxt = 1 - b
            wait_store(nxt)
            start_gather(g + 1, nxt)
            wait_gather(b)
            start_store(g, b)
        return carry

    lax.fori_loop(0, (NCHUNK - 2) // 2, steady, 0)

    wait_gather(1)
    start_store(NCHUNK - 1, 1)
    wait_store(0)
    wait_store(1)


def kernel(embed_id, weight):
    idx = embed_id.astype(jnp.int32).reshape(NUM_WORKERS, NCHUNK, CHUNK)
    return _codebook_gather(weight, idx)
